# Initial kernel scaffold; baseline (speedup 1.0000x reference)
import jax
import jax.numpy as jnp
from jax.experimental import pallas as pl


def kernel(x, ew1, eb1, ew2, eb2, ew3, eb3, emb, dw1, db1, dw2, db2, dw3, db3):
    raise NotImplementedError("write your pallas kernel here")


# speedup vs baseline: 2.5634x; 2.5634x over previous
import functools
import jax
import jax.numpy as jnp
from jax import lax
from jax.experimental import pallas as pl
from jax.experimental.pallas import tpu as pltpu

F32 = jnp.float32

_TAPS = {0: ((1, 1), (0, 3)), 1: ((1, 2), (2, 0))}
_PH = {0: ((1, 0), (0, 1), (1, 1)), 1: ((0, 1), (1, 1), (0, 2))}


def _nhwc(x):
    return x.transpose(0, 2, 3, 1)


def _pad_hw(x, p):
    return jnp.pad(x, ((0, 0), (p, p), (p, p), (0, 0)))


def _phases(x):
    n, h2, w2, c = x.shape
    x = x.reshape(n, h2 // 2, 2, w2 // 2, 2, c)
    return x.transpose(2, 4, 0, 1, 3, 5)


def _interleave(ph):
    n, _, _, h, w, c = ph.shape
    return ph.transpose(0, 3, 1, 4, 2, 5).reshape(n, 2 * h, 2 * w, c)


def _conv1_body(xc_ref, w_ref, b_ref, o_ref, *, ho, rc):
    co = o_ref.shape[-1]
    kp = xc_ref.shape[-1]
    for r0 in range(0, ho, rc):
        m = rc * ho
        acc = jnp.zeros((m, co), F32)
        for dh in range(2):
            for dw in range(2):
                xt = xc_ref[0, r0 + dh:r0 + dh + rc, dw:dw + ho, :]
                xt = xt.reshape(m, kp)
                acc = acc + jnp.dot(xt, w_ref[dh, dw],
                                    preferred_element_type=F32)
        acc = jnp.maximum(acc + b_ref[:], 0.0)
        o_ref[0, r0:r0 + rc] = acc.reshape(rc, ho, co)


def _conv1(x_nhwc, w_oihw, b, *, rc):
    n, h, _, ci = x_nhwc.shape
    co = w_oihw.shape[0]
    ho = h // 2
    if ho % rc:
        rc = ho
    pp = _phases(_pad_hw(x_nhwc, 1))
    hp = ho + 1
    xc = pp.transpose(2, 3, 4, 0, 1, 5).reshape(n, hp, hp, 4 * ci)
    wt = w_oihw.transpose(2, 3, 1, 0)
    w12 = wt.reshape(2, 2, 2, 2, ci, co).transpose(0, 2, 1, 3, 4, 5)
    w12 = w12.reshape(2, 2, 4 * ci, co)
    return pl.pallas_call(
        functools.partial(_conv1_body, ho=ho, rc=rc),
        grid=(n,),
        in_specs=[
            pl.BlockSpec((1, hp, hp, 4 * ci), lambda i: (i, 0, 0, 0)),
            pl.BlockSpec((2, 2, 4 * ci, co), lambda i: (0, 0, 0, 0)),
            pl.BlockSpec((co,), lambda i: (0,)),
        ],
        out_specs=pl.BlockSpec((1, ho, ho, co), lambda i: (i, 0, 0, 0)),
        out_shape=jax.ShapeDtypeStruct((n, ho, ho, co), F32),
    )(xc, w12, b)


def _conv_s2_body(pp_ref, wt_ref, b_ref, o_ref, *, ho, rc):
    ci = pp_ref.shape[-1]
    co = o_ref.shape[-1]
    for r0 in range(0, ho, rc):
        acc = jnp.zeros((rc * ho, co), F32)
        for a in range(2):
            for b_ in range(2):
                for dh in range(2):
                    for dw in range(2):
                        xt = pp_ref[a, b_, 0, r0 + dh:r0 + dh + rc,
                                    dw:dw + ho, :]
                        xt = xt.reshape(rc * ho, ci)
                        acc = acc + jnp.dot(xt, wt_ref[dh, a, dw, b_],
                                            preferred_element_type=F32)
        acc = jnp.maximum(acc + b_ref[:], 0.0)
        o_ref[0, r0:r0 + rc] = acc.reshape(rc, ho, co)


def _conv_s2(x_nhwc, w_oihw, b, *, rc):
    n, h, _, ci = x_nhwc.shape
    co = w_oihw.shape[0]
    ho = h // 2
    if ho % rc:
        rc = ho
    pp = _phases(_pad_hw(x_nhwc, 1))
    hp = ho + 1
    wt = w_oihw.transpose(2, 3, 1, 0).reshape(2, 2, 2, 2, ci, co)
    return pl.pallas_call(
        functools.partial(_conv_s2_body, ho=ho, rc=rc),
        grid=(n,),
        in_specs=[
            pl.BlockSpec((2, 2, 1, hp, hp, ci), lambda i: (0, 0, i, 0, 0, 0)),
            pl.BlockSpec((2, 2, 2, 2, ci, co), lambda i: (0, 0, 0, 0, 0, 0)),
            pl.BlockSpec((co,), lambda i: (0,)),
        ],
        out_specs=pl.BlockSpec((1, ho, ho, co), lambda i: (i, 0, 0, 0)),
        out_shape=jax.ShapeDtypeStruct((n, ho, ho, co), F32),
    )(pp, wt, b)


def _conv3_vq_body(zp_ref, wt_ref, b_ref, embt_ref, emb_ref, zq_ref, idx_ref,
                   *, ho, rc):
    ci = zp_ref.shape[-1]
    nv = emb_ref.shape[0]
    en = jnp.sum(embt_ref[:] * embt_ref[:], axis=0)
    for r0 in range(0, ho, rc):
        m = rc * ho
        acc = jnp.zeros((m, ci), F32)
        for kh in range(3):
            for kw in range(3):
                xt = zp_ref[0, r0 + kh:r0 + kh + rc, kw:kw + ho, :]
                xt = xt.reshape(m, ci)
                acc = acc + jnp.dot(xt, wt_ref[kh, kw],
                                    preferred_element_type=F32)
        z = acc + b_ref[:]
        scores = en[None, :] - 2.0 * jnp.dot(z, embt_ref[:],
                                             preferred_element_type=F32)
        mins = jnp.min(scores, axis=1, keepdims=True)
        iota = lax.broadcasted_iota(jnp.int32, (m, nv), 1)
        idx = jnp.min(jnp.where(scores <= mins, iota, nv), axis=1)
        onehot = (iota == idx[:, None]).astype(F32)
        zq = jnp.dot(onehot, emb_ref[:], preferred_element_type=F32)
        zq_ref[0, r0:r0 + rc] = zq.reshape(rc, ho, ci)
        idx_ref[0, r0 * ho:(r0 + rc) * ho] = idx[:, None]


def _conv3_vq(x_nhwc, w_oihw, b, emb, *, rc):
    n, h, _, ci = x_nhwc.shape
    nv = emb.shape[0]
    if h % rc:
        rc = h
    zp = _pad_hw(x_nhwc, 1)
    wt = w_oihw.transpose(2, 3, 1, 0)
    embt = emb.T
    return pl.pallas_call(
        functools.partial(_conv3_vq_body, ho=h, rc=rc),
        grid=(n,),
        in_specs=[
            pl.BlockSpec((1, h + 2, h + 2, ci), lambda i: (i, 0, 0, 0)),
            pl.BlockSpec((3, 3, ci, ci), lambda i: (0, 0, 0, 0)),
            pl.BlockSpec((ci,), lambda i: (0,)),
            pl.BlockSpec((ci, nv), lambda i: (0, 0)),
            pl.BlockSpec((nv, ci), lambda i: (0, 0)),
        ],
        out_specs=[
            pl.BlockSpec((1, h, h, ci), lambda i: (i, 0, 0, 0)),
            pl.BlockSpec((1, h * h, 1), lambda i: (i, 0, 0)),
        ],
        out_shape=[
            jax.ShapeDtypeStruct((n, h, h, ci), F32),
            jax.ShapeDtypeStruct((n, h * h, 1), jnp.int32),
        ],
    )(zp, wt, b, embt, emb)


def _deconv1_body(zp_ref, wt_ref, b_ref, o_ref, *, ho, rc):
    ci = zp_ref.shape[-1]
    co = o_ref.shape[-1]
    for a in range(2):
        for b_ in range(2):
            for r0 in range(0, ho, rc):
                m = rc * ho
                acc = jnp.zeros((m, co), F32)
                for (dr, kh) in _TAPS[a]:
                    for (dc, kw) in _TAPS[b_]:
                        xt = zp_ref[0, r0 + dr:r0 + dr + rc, dc:dc + ho, :]
                        xt = xt.reshape(m, ci)
                        acc = acc + jnp.dot(xt, wt_ref[kh, kw],
                                            preferred_element_type=F32)
                acc = jnp.maximum(acc + b_ref[:], 0.0)
                o_ref[0, a, b_, r0:r0 + rc] = acc.reshape(rc, ho, co)


def _deconv1(x_nhwc, w_iokk, b, *, rc):
    n, h, _, ci = x_nhwc.shape
    co = w_iokk.shape[1]
    if h % rc:
        rc = h
    zp = _pad_hw(x_nhwc, 1)
    wt = w_iokk.transpose(2, 3, 0, 1)
    ph = pl.pallas_call(
        functools.partial(_deconv1_body, ho=h, rc=rc),
        grid=(n,),
        in_specs=[
            pl.BlockSpec((1, h + 2, h + 2, ci), lambda i: (i, 0, 0, 0)),
            pl.BlockSpec((4, 4, ci, co), lambda i: (0, 0, 0, 0)),
            pl.BlockSpec((co,), lambda i: (0,)),
        ],
        out_specs=pl.BlockSpec((1, 2, 2, h, h, co),
                               lambda i: (i, 0, 0, 0, 0, 0)),
        out_shape=jax.ShapeDtypeStruct((n, 2, 2, h, h, co), F32),
    )(zp, wt, b)
    return _interleave(ph)


def _deconv2_body(zp_ref, wc_ref, b_ref, o_ref, *, ho, rc):
    ci = zp_ref.shape[-1]
    cn = o_ref.shape[-1]
    for a in range(2):
        for r0 in range(0, ho, rc):
            m = rc * ho
            acc = jnp.zeros((m, cn), F32)
            for (dr, kh) in _TAPS[a]:
                for dc in range(3):
                    xt = zp_ref[0, r0 + dr:r0 + dr + rc, dc:dc + ho, :]
                    xt = xt.reshape(m, ci)
                    acc = acc + jnp.dot(xt, wc_ref[kh, dc],
                                        preferred_element_type=F32)
            acc = jnp.maximum(acc + b_ref[:], 0.0)
            o_ref[0, a, r0:r0 + rc] = acc.reshape(rc, ho, cn)


def _deconv2(x_nhwc, w_iokk, b, *, rc):
    n, h, _, ci = x_nhwc.shape
    co = w_iokk.shape[1]
    if h % rc:
        rc = h
    zp = _pad_hw(x_nhwc, 1)
    wt = w_iokk.transpose(2, 3, 0, 1)
    zb = jnp.zeros((ci, co), F32)
    colw = {0: (3, None), 1: (1, 2), 2: (None, 0)}
    wc = jnp.stack([
        jnp.stack([
            jnp.concatenate(
                [wt[kh, colw[dc][0]] if colw[dc][0] is not None else zb,
                 wt[kh, colw[dc][1]] if colw[dc][1] is not None else zb],
                axis=1)
            for dc in range(3)], axis=0)
        for kh in range(4)], axis=0)
    b2 = jnp.concatenate([b, b])
    return pl.pallas_call(
        functools.partial(_deconv2_body, ho=h, rc=rc),
        grid=(n,),
        in_specs=[
            pl.BlockSpec((1, h + 2, h + 2, ci), lambda i: (i, 0, 0, 0)),
            pl.BlockSpec((4, 3, ci, 2 * co), lambda i: (0, 0, 0, 0)),
            pl.BlockSpec((2 * co,), lambda i: (0,)),
        ],
        out_specs=pl.BlockSpec((1, 2, h, h, 2 * co),
                               lambda i: (i, 0, 0, 0, 0)),
        out_shape=jax.ShapeDtypeStruct((n, 2, h, h, 2 * co), F32),
    )(zp, wc, b2)


_ROWTAPS = ((1, 0), (0, 1), (1, 1), (0, 2))


def _conv4_body(xq_ref, w_ref, b_ref, o_ref, *, ho, rc):
    kp = xq_ref.shape[-1]
    cn = o_ref.shape[-1]
    for r0 in range(0, ho, rc):
        m = rc * ho
        acc = jnp.zeros((m, cn), F32)
        for t, (pr, sr) in enumerate(_ROWTAPS):
            for sc in range(3):
                xt = xq_ref[0, pr, r0 + sr:r0 + sr + rc, sc:sc + ho, :]
                xt = xt.reshape(m, kp)
                acc = acc + jnp.dot(xt, w_ref[t, sc],
                                    preferred_element_type=F32)
        o_ref[0, r0:r0 + rc] = jnp.tanh(acc + b_ref[:]).reshape(rc, ho, cn)


def _conv4(d2q, w_oihw, b, *, rc):
    n, _, hp2, _, kp = d2q.shape
    h = hp2 - 2
    ci = kp // 2
    co = w_oihw.shape[0]
    if h % rc:
        rc = h
    w3 = w_oihw.transpose(2, 3, 1, 0)
    blocks = []
    for (pr, sr) in _ROWTAPS:
        row = []
        for sc in range(3):
            mat = jnp.zeros((2 * ci, 4 * co), F32)
            for al in range(2):
                for kh in range(3):
                    if _PH[al][kh] != (pr, sr):
                        continue
                    for be in range(2):
                        for kw in range(3):
                            pc, scc = _PH[be][kw]
                            if scc != sc:
                                continue
                            mat = mat.at[ci * pc:ci * (pc + 1),
                                         (2 * al + be) * co:
                                         (2 * al + be + 1) * co].set(w3[kh, kw])
            row.append(mat)
        blocks.append(jnp.stack(row, axis=0))
    w12 = jnp.stack(blocks, axis=0)
    b12 = jnp.tile(b, 4)
    return pl.pallas_call(
        functools.partial(_conv4_body, ho=h, rc=rc),
        grid=(n,),
        in_specs=[
            pl.BlockSpec((1, 2, hp2, hp2, kp), lambda i: (i, 0, 0, 0, 0)),
            pl.BlockSpec((4, 3, kp, 4 * co), lambda i: (0, 0, 0, 0)),
            pl.BlockSpec((4 * co,), lambda i: (0,)),
        ],
        out_specs=pl.BlockSpec((1, h, h, 4 * co), lambda i: (i, 0, 0, 0)),
        out_shape=jax.ShapeDtypeStruct((n, h, h, 4 * co), F32),
    )(d2q, w12, b12)


def kernel(x, ew1, eb1, ew2, eb2, ew3, eb3, emb, dw1, db1, dw2, db2, dw3, db3):
    h = _conv1(_nhwc(x), ew1, eb1, rc=16)
    h = _conv_s2(h, ew2, eb2, rc=28)
    zq, _idx = _conv3_vq(h, ew3, eb3, emb, rc=28)
    d = _deconv1(zq, dw1, db1, rc=28)
    d2 = _deconv2(d, dw2, db2, rc=28)
    d2q = jnp.pad(d2, ((0, 0), (0, 0), (1, 1), (1, 1), (0, 0)))
    y12 = _conv4(d2q, dw3, db3, rc=28)
    n, hh, _, _ = y12.shape
    co = dw3.shape[0]
    y = y12.reshape(n, hh, hh, 2, 2, co)
    y = y.transpose(0, 5, 1, 3, 2, 4).reshape(n, co, 2 * hh, 2 * hh)
    return y
